# baseline (speedup 1.0000x reference)
import functools
import math

import jax
import jax.numpy as jnp
from jax import lax
from jax.experimental import pallas as pl
from jax.experimental.pallas import tpu as pltpu
from jax.experimental.pallas import tpu_sc as plsc

N = 10000
M = 32
F = 128
K = 16
NM = N * M

NC, NS = 2, 16
NODES_PER_GROUP = 8

C0 = 0.28209479177387814
SCALE = C0 / (M * math.sqrt(float(F)))

LB = 16000


_PERM = []
for _q in range(4):
    for _t in range(16):
        _PERM.append(32 * _q + _t)
        _PERM.append(32 * _q + 16 + _t)
TCB = 2000


def _prep_body(a_ref, x_ref, wtp_ref, w1b_ref, b1b_ref, w2b_ref, b2_ref,
               bout_ref, rout_ref):
    bout_ref[...] = jnp.dot(a_ref[...], wtp_ref[...],
                            preferred_element_type=jnp.float32) * SCALE
    h = jnp.dot(x_ref[...], w1b_ref[...],
                preferred_element_type=jnp.float32) + b1b_ref[...]
    sp = jnp.maximum(h, 0.0) + jnp.log1p(jnp.exp(-jnp.abs(h)))
    rout_ref[...] = jnp.dot(sp, w2b_ref[...],
                            preferred_element_type=jnp.float32) + b2_ref[...]


def _prep(atom_fea, nbr2, wtp, w1big, b1big, w2big, b2s):
    return pl.pallas_call(
        _prep_body,
        grid=(N // TCB,),
        in_specs=[pl.BlockSpec((TCB, F), lambda i: (i, 0)),
                  pl.BlockSpec((TCB, M * K), lambda i: (i, 0)),
                  pl.BlockSpec((F, F), lambda i: (0, 0)),
                  pl.BlockSpec((M * K, M * K), lambda i: (0, 0)),
                  pl.BlockSpec((1, M * K), lambda i: (0, 0)),
                  pl.BlockSpec((M * K, M), lambda i: (0, 0)),
                  pl.BlockSpec((1, 1), lambda i: (0, 0))],
        out_specs=[pl.BlockSpec((TCB, F), lambda i: (i, 0)),
                   pl.BlockSpec((TCB, M), lambda i: (i, 0))],
        out_shape=[jax.ShapeDtypeStruct((N, F), jnp.float32),
                   jax.ShapeDtypeStruct((N, M), jnp.float32)],
    )(atom_fea, nbr2, wtp, w1big, b1big, w2big, b2s)


_GDN = lax.GatherDimensionNumbers(offset_dims=(), collapsed_slice_dims=(0,),
                                  start_index_map=(0,))


def _splat(vec, lane):
    idx = jnp.full((16, 1), lane, jnp.int32)
    return lax.gather(vec, idx, _GDN, slice_sizes=(1,),
                      mode=lax.GatherScatterMode.PROMISE_IN_BOUNDS)


G0, R0 = 38, 9
G1, R1 = 39, 9
T0 = NS * G0 + R0
NGROUPS = N // NODES_PER_GROUP
assert T0 + NS * G1 + R1 == NGROUPS
GMAX = max(G0, G1) + 1
SLAB_E = GMAX * NODES_PER_GROUP * M


def _sc_body(b_hbm, idx_hbm, r_hbm, out_hbm, idx_v, r_v, rows0, rows1,
             out_v, sem0, sem1, sem_out):
    c = lax.axis_index("c")
    s = lax.axis_index("s")
    ngroups = jnp.where(c == 0, G0 + (s < R0), G1 + (s < R1))
    base = jnp.where(c == 0, s * G0 + jnp.minimum(s, R0),
                     T0 + s * G1 + jnp.minimum(s, R1))
    nh = 2 * ngroups
    eb = base * (NODES_PER_GROUP * M)
    start = pl.multiple_of(jnp.minimum(eb, NM - SLAB_E), 8)
    off = eb - start
    pltpu.sync_copy(idx_hbm.at[pl.ds(start, SLAB_E)], idx_v)
    pltpu.sync_copy(r_hbm.at[pl.ds(start, SLAB_E)], r_v)
    rows = (rows0, rows1)
    sems = (sem0, sem1)
    pltpu.async_copy(b_hbm.at[idx_v.at[pl.ds(off, 128)]], rows0, sem0)

    def group(g, carry):
        gg = base + g
        out_slice = out_hbm.at[pl.ds(
            pl.multiple_of(gg * NODES_PER_GROUP, NODES_PER_GROUP),
            NODES_PER_GROUP)]

        @pl.when(g > 0)
        def _drain_out():
            pltpu.make_async_copy(out_v, out_slice, sem_out).wait()

        for p in range(2):
            h = g * 2 + p
            le = off + h * 128

            @pl.when(h < nh - 1)
            def _issue_next():
                pltpu.async_copy(b_hbm.at[idx_v.at[pl.ds(le + 128, 128)]],
                                 rows[1 - p], sems[1 - p])

            pltpu.make_async_copy(b_hbm.at[idx_v.at[pl.ds(off, 128)]],
                                  rows[p], sems[p]).wait()
            for nn in range(4):
                acc0 = [jnp.zeros((16,), jnp.float32) for _ in range(F // 16)]
                acc1 = [jnp.zeros((16,), jnp.float32) for _ in range(F // 16)]
                for jh in range(2):
                    r16 = r_v[pl.ds(le + nn * M + jh * 16, 16)]
                    for jj in range(8):
                        e0 = nn * M + jh * 16 + 2 * jj
                        ra = _splat(r16, 2 * jj)
                        rb = _splat(r16, 2 * jj + 1)
                        for f in range(F // 16):
                            acc0[f] = acc0[f] + ra * rows[p][e0, pl.ds(f * 16, 16)]
                            acc1[f] = acc1[f] + rb * rows[p][e0 + 1,
                                                             pl.ds(f * 16, 16)]
                n = p * 4 + nn
                for f in range(F // 16):
                    out_v[n, pl.ds(f * 16, 16)] = acc0[f] + acc1[f]
        pltpu.async_copy(out_v, out_slice, sem_out)
        return carry

    lax.fori_loop(0, ngroups, group, 0)
    final = out_hbm.at[pl.ds((base + ngroups - 1) * NODES_PER_GROUP,
                             NODES_PER_GROUP)]
    pltpu.make_async_copy(out_v, final, sem_out).wait()


_sc_mesh = plsc.VectorSubcoreMesh(core_axis_name="c", subcore_axis_name="s",
                                  num_cores=NC, num_subcores=NS)

_sc_gather = functools.partial(
    pl.kernel,
    out_type=jax.ShapeDtypeStruct((N, F), jnp.float32),
    mesh=_sc_mesh,
    scratch_types=[
        pltpu.VMEM((SLAB_E,), jnp.int32),
        pltpu.VMEM((SLAB_E,), jnp.float32),
        pltpu.VMEM((128, F), jnp.float32),
        pltpu.VMEM((128, F), jnp.float32),
        pltpu.VMEM((NODES_PER_GROUP, F), jnp.float32),
        pltpu.SemaphoreType.DMA,
        pltpu.SemaphoreType.DMA,
        pltpu.SemaphoreType.DMA,
    ],
)(_sc_body)


def kernel(atom_fea, nbr_fea, nbr_idx, pos, W1, b1, W2, b2, Wtp):
    del pos
    eye = jnp.eye(M, dtype=jnp.float32)
    mask1 = jnp.repeat(jnp.repeat(eye, K, axis=0), K, axis=1)
    w1big = mask1 * jnp.tile(W1, (M, M))
    mask2 = jnp.repeat(eye, K, axis=0)
    w2big = mask2 * jnp.tile(W2[:, :1], (M, M))
    b1big = jnp.tile(b1, M).reshape(1, M * K)
    bmat, r = _prep(atom_fea, nbr_fea.reshape(N, M * K), Wtp,
                    w1big, b1big, w2big, b2[0].reshape(1, 1))
    out = _sc_gather(bmat, nbr_idx.astype(jnp.int32).reshape(NM),
                     r.reshape(NM))
    return out


# speedup vs baseline: 1.0528x; 1.0528x over previous
import functools
import math

import jax
import jax.numpy as jnp
from jax import lax
from jax.experimental import pallas as pl
from jax.experimental.pallas import tpu as pltpu
from jax.experimental.pallas import tpu_sc as plsc

N = 10000
M = 32
F = 128
K = 16
NM = N * M

NC, NS = 2, 16
NODES_PER_GROUP = 8

C0 = 0.28209479177387814
SCALE = C0 / (M * math.sqrt(float(F)))

LB = 16000


_PERM = []
for _q in range(4):
    for _t in range(16):
        _PERM.append(32 * _q + _t)
        _PERM.append(32 * _q + 16 + _t)
TCB = 2000


def _prep_body(a_ref, x_ref, wtp_ref, w1b_ref, b1b_ref, w2b_ref, b2_ref,
               bout_ref, rout_ref):
    bout_ref[...] = jnp.dot(a_ref[...], wtp_ref[...],
                            preferred_element_type=jnp.float32) * SCALE
    h = jnp.dot(x_ref[...], w1b_ref[...],
                preferred_element_type=jnp.float32) + b1b_ref[...]
    sp = jnp.maximum(h, 0.0) + jnp.log1p(jnp.exp(-jnp.abs(h)))
    rout_ref[...] = jnp.dot(sp, w2b_ref[...],
                            preferred_element_type=jnp.float32) + b2_ref[...]


def _prep(atom_fea, nbr2, wtp, w1big, b1big, w2big, b2s):
    return pl.pallas_call(
        _prep_body,
        grid=(N // TCB,),
        in_specs=[pl.BlockSpec((TCB, F), lambda i: (i, 0)),
                  pl.BlockSpec((TCB, M * K), lambda i: (i, 0)),
                  pl.BlockSpec((F, F), lambda i: (0, 0)),
                  pl.BlockSpec((M * K, M * K), lambda i: (0, 0)),
                  pl.BlockSpec((1, M * K), lambda i: (0, 0)),
                  pl.BlockSpec((M * K, M), lambda i: (0, 0)),
                  pl.BlockSpec((1, 1), lambda i: (0, 0))],
        out_specs=[pl.BlockSpec((TCB, F), lambda i: (i, 0)),
                   pl.BlockSpec((TCB, M), lambda i: (i, 0))],
        out_shape=[jax.ShapeDtypeStruct((N, F), jnp.float32),
                   jax.ShapeDtypeStruct((N, M), jnp.float32)],
    )(atom_fea, nbr2, wtp, w1big, b1big, w2big, b2s)


_GDN = lax.GatherDimensionNumbers(offset_dims=(), collapsed_slice_dims=(0,),
                                  start_index_map=(0,))


def _splat(vec, lane):
    idx = jnp.full((16, 1), lane, jnp.int32)
    return lax.gather(vec, idx, _GDN, slice_sizes=(1,),
                      mode=lax.GatherScatterMode.PROMISE_IN_BOUNDS)


G0, R0 = 38, 9
G1, R1 = 39, 9
T0 = NS * G0 + R0
NGROUPS = N // NODES_PER_GROUP
assert T0 + NS * G1 + R1 == NGROUPS
GMAX = max(G0, G1) + 1
SLAB_E = GMAX * NODES_PER_GROUP * M


def _sc_body(b_hbm, idx_hbm, r_hbm, out_hbm, idx_v, r_v, rows0, rows1,
             out_v, sem0, sem1, sem_out):
    c = lax.axis_index("c")
    s = lax.axis_index("s")
    ngroups = jnp.where(c == 0, G0 + (s < R0), G1 + (s < R1))
    base = jnp.where(c == 0, s * G0 + jnp.minimum(s, R0),
                     T0 + s * G1 + jnp.minimum(s, R1))
    nh = 2 * ngroups
    eb = base * (NODES_PER_GROUP * M)
    start = pl.multiple_of(jnp.minimum(eb, NM - SLAB_E), 8)
    off = eb - start
    pltpu.sync_copy(idx_hbm.at[pl.ds(start, SLAB_E)], idx_v)
    pltpu.sync_copy(r_hbm.at[pl.ds(start, SLAB_E)], r_v)
    rows = (rows0, rows1)
    sems = (sem0, sem1)
    pltpu.async_copy(b_hbm.at[idx_v.at[pl.ds(off, 128)]], rows0, sem0)

    def group(g, carry):
        gg = base + g
        out_slice = out_hbm.at[pl.ds(
            pl.multiple_of(gg * NODES_PER_GROUP, NODES_PER_GROUP),
            NODES_PER_GROUP)]

        @pl.when(g > 0)
        def _drain_out():
            pltpu.make_async_copy(out_v, out_slice, sem_out).wait()

        for p in range(2):
            h = g * 2 + p
            le = off + h * 128

            @pl.when(h < nh - 1)
            def _issue_next():
                pltpu.async_copy(b_hbm.at[idx_v.at[pl.ds(le + 128, 128)]],
                                 rows[1 - p], sems[1 - p])

            pltpu.make_async_copy(b_hbm.at[idx_v.at[pl.ds(off, 128)]],
                                  rows[p], sems[p]).wait()
            for nn in range(4):
                accs = [jnp.zeros((16,), jnp.float32) for _ in range(F // 16)]
                nf = F // 16
                e0 = nn * M
                r16 = r_v[pl.ds(le + e0, 16)]
                vals = [rows[p][e0, pl.ds(f * 16, 16)] for f in range(nf)]
                rs = _splat(r16, 0)
                for j in range(1, M + 1):
                    if j < M:
                        e = e0 + j
                        if j == 16:
                            r16 = r_v[pl.ds(le + e0 + 16, 16)]
                        nvals = [rows[p][e, pl.ds(f * 16, 16)]
                                 for f in range(nf)]
                        nrs = _splat(r16, j % 16)
                    for f in range(nf):
                        accs[f] = accs[f] + rs * vals[f]
                    if j < M:
                        vals, rs = nvals, nrs
                n = p * 4 + nn
                for f in range(nf):
                    out_v[n, pl.ds(f * 16, 16)] = accs[f]
        pltpu.async_copy(out_v, out_slice, sem_out)
        return carry

    lax.fori_loop(0, ngroups, group, 0)
    final = out_hbm.at[pl.ds((base + ngroups - 1) * NODES_PER_GROUP,
                             NODES_PER_GROUP)]
    pltpu.make_async_copy(out_v, final, sem_out).wait()


_sc_mesh = plsc.VectorSubcoreMesh(core_axis_name="c", subcore_axis_name="s",
                                  num_cores=NC, num_subcores=NS)

_sc_gather = functools.partial(
    pl.kernel,
    out_type=jax.ShapeDtypeStruct((N, F), jnp.float32),
    mesh=_sc_mesh,
    scratch_types=[
        pltpu.VMEM((SLAB_E,), jnp.int32),
        pltpu.VMEM((SLAB_E,), jnp.float32),
        pltpu.VMEM((128, F), jnp.float32),
        pltpu.VMEM((128, F), jnp.float32),
        pltpu.VMEM((NODES_PER_GROUP, F), jnp.float32),
        pltpu.SemaphoreType.DMA,
        pltpu.SemaphoreType.DMA,
        pltpu.SemaphoreType.DMA,
    ],
)(_sc_body)


def kernel(atom_fea, nbr_fea, nbr_idx, pos, W1, b1, W2, b2, Wtp):
    del pos
    eye = jnp.eye(M, dtype=jnp.float32)
    mask1 = jnp.repeat(jnp.repeat(eye, K, axis=0), K, axis=1)
    w1big = mask1 * jnp.tile(W1, (M, M))
    mask2 = jnp.repeat(eye, K, axis=0)
    w2big = mask2 * jnp.tile(W2[:, :1], (M, M))
    b1big = jnp.tile(b1, M).reshape(1, M * K)
    bmat, r = _prep(atom_fea, nbr_fea.reshape(N, M * K), Wtp,
                    w1big, b1big, w2big, b2[0].reshape(1, 1))
    out = _sc_gather(bmat, nbr_idx.astype(jnp.int32).reshape(NM),
                     r.reshape(NM))
    return out


# speedup vs baseline: 1.1083x; 1.0527x over previous
import functools
import math

import jax
import jax.numpy as jnp
from jax import lax
from jax.experimental import pallas as pl
from jax.experimental.pallas import tpu as pltpu
from jax.experimental.pallas import tpu_sc as plsc

N = 10000
M = 32
F = 128
K = 16
NM = N * M

NC, NS = 2, 16
NODES_PER_GROUP = 8

C0 = 0.28209479177387814
SCALE = C0 / (M * math.sqrt(float(F)))

LB = 16000


_PERM = []
for _q in range(4):
    for _t in range(16):
        _PERM.append(32 * _q + _t)
        _PERM.append(32 * _q + 16 + _t)
TCB = 2000


def _prep_body(a_ref, x_ref, wtp_ref, w1b_ref, b1b_ref, w2b_ref, b2_ref,
               bout_ref, rout_ref):
    bout_ref[...] = jnp.dot(a_ref[...], wtp_ref[...],
                            preferred_element_type=jnp.float32) * SCALE
    h = jnp.dot(x_ref[...], w1b_ref[...],
                preferred_element_type=jnp.float32) + b1b_ref[...]
    sp = jnp.maximum(h, 0.0) + jnp.log1p(jnp.exp(-jnp.abs(h)))
    rout_ref[...] = jnp.dot(sp, w2b_ref[...],
                            preferred_element_type=jnp.float32) + b2_ref[...]


def _prep(atom_fea, nbr2, wtp, w1big, b1big, w2big, b2s):
    return pl.pallas_call(
        _prep_body,
        grid=(N // TCB,),
        in_specs=[pl.BlockSpec((TCB, F), lambda i: (i, 0)),
                  pl.BlockSpec((TCB, M * K), lambda i: (i, 0)),
                  pl.BlockSpec((F, F), lambda i: (0, 0)),
                  pl.BlockSpec((M * K, M * K), lambda i: (0, 0)),
                  pl.BlockSpec((1, M * K), lambda i: (0, 0)),
                  pl.BlockSpec((M * K, M), lambda i: (0, 0)),
                  pl.BlockSpec((1, 1), lambda i: (0, 0))],
        out_specs=[pl.BlockSpec((TCB, F), lambda i: (i, 0)),
                   pl.BlockSpec((TCB, M), lambda i: (i, 0))],
        out_shape=[jax.ShapeDtypeStruct((N, F), jnp.float32),
                   jax.ShapeDtypeStruct((N, M), jnp.float32)],
    )(atom_fea, nbr2, wtp, w1big, b1big, w2big, b2s)


_GDN = lax.GatherDimensionNumbers(offset_dims=(), collapsed_slice_dims=(0,),
                                  start_index_map=(0,))


def _splat(vec, lane):
    idx = jnp.full((16, 1), lane, jnp.int32)
    return lax.gather(vec, idx, _GDN, slice_sizes=(1,),
                      mode=lax.GatherScatterMode.PROMISE_IN_BOUNDS)


G0, R0 = 39, 6
G1, R1 = 38, 12
T0 = NS * G0 + R0
NGROUPS = N // NODES_PER_GROUP
assert T0 + NS * G1 + R1 == NGROUPS
GMAX = max(G0, G1) + 1
SLAB_E = GMAX * NODES_PER_GROUP * M


def _sc_body(b_hbm, idx_hbm, r_hbm, out_hbm, idx_v, r_v, rows0, rows1,
             out_v, sem0, sem1, sem_out):
    c = lax.axis_index("c")
    s = lax.axis_index("s")
    ngroups = jnp.where(c == 0, G0 + (s < R0), G1 + (s < R1))
    base = jnp.where(c == 0, s * G0 + jnp.minimum(s, R0),
                     T0 + s * G1 + jnp.minimum(s, R1))
    nh = 2 * ngroups
    eb = base * (NODES_PER_GROUP * M)
    start = pl.multiple_of(jnp.minimum(eb, NM - SLAB_E), 8)
    off = eb - start
    pltpu.sync_copy(idx_hbm.at[pl.ds(start, SLAB_E)], idx_v)
    pltpu.sync_copy(r_hbm.at[pl.ds(start, SLAB_E)], r_v)
    rows = (rows0, rows1)
    sems = (sem0, sem1)
    pltpu.async_copy(b_hbm.at[idx_v.at[pl.ds(off, 128)]], rows0, sem0)

    def group(g, carry):
        gg = base + g
        out_slice = out_hbm.at[pl.ds(
            pl.multiple_of(gg * NODES_PER_GROUP, NODES_PER_GROUP),
            NODES_PER_GROUP)]

        @pl.when(g > 0)
        def _drain_out():
            pltpu.make_async_copy(out_v, out_slice, sem_out).wait()

        for p in range(2):
            h = g * 2 + p
            le = off + h * 128

            @pl.when(h < nh - 1)
            def _issue_next():
                pltpu.async_copy(b_hbm.at[idx_v.at[pl.ds(le + 128, 128)]],
                                 rows[1 - p], sems[1 - p])

            pltpu.make_async_copy(b_hbm.at[idx_v.at[pl.ds(off, 128)]],
                                  rows[p], sems[p]).wait()
            for nn in range(4):
                accs = [jnp.zeros((16,), jnp.float32) for _ in range(F // 16)]
                nf = F // 16
                e0 = nn * M
                r16 = r_v[pl.ds(le + e0, 16)]
                vals = [rows[p][e0, pl.ds(f * 16, 16)] for f in range(nf)]
                rs = _splat(r16, 0)
                for j in range(1, M + 1):
                    if j < M:
                        e = e0 + j
                        if j == 16:
                            r16 = r_v[pl.ds(le + e0 + 16, 16)]
                        nvals = [rows[p][e, pl.ds(f * 16, 16)]
                                 for f in range(nf)]
                        nrs = _splat(r16, j % 16)
                    for f in range(nf):
                        accs[f] = accs[f] + rs * vals[f]
                    if j < M:
                        vals, rs = nvals, nrs
                n = p * 4 + nn
                for f in range(nf):
                    out_v[n, pl.ds(f * 16, 16)] = accs[f]
        pltpu.async_copy(out_v, out_slice, sem_out)
        return carry

    lax.fori_loop(0, ngroups, group, 0)
    final = out_hbm.at[pl.ds((base + ngroups - 1) * NODES_PER_GROUP,
                             NODES_PER_GROUP)]
    pltpu.make_async_copy(out_v, final, sem_out).wait()


_sc_mesh = plsc.VectorSubcoreMesh(core_axis_name="c", subcore_axis_name="s",
                                  num_cores=NC, num_subcores=NS)

_sc_gather = functools.partial(
    pl.kernel,
    out_type=jax.ShapeDtypeStruct((N, F), jnp.float32),
    mesh=_sc_mesh,
    scratch_types=[
        pltpu.VMEM((SLAB_E,), jnp.int32),
        pltpu.VMEM((SLAB_E,), jnp.float32),
        pltpu.VMEM((128, F), jnp.float32),
        pltpu.VMEM((128, F), jnp.float32),
        pltpu.VMEM((NODES_PER_GROUP, F), jnp.float32),
        pltpu.SemaphoreType.DMA,
        pltpu.SemaphoreType.DMA,
        pltpu.SemaphoreType.DMA,
    ],
)(_sc_body)


def kernel(atom_fea, nbr_fea, nbr_idx, pos, W1, b1, W2, b2, Wtp):
    del pos
    a512 = jnp.arange(M * K, dtype=jnp.int32)
    proj = (a512[:, None] % K == jnp.arange(K)[None, :]).astype(jnp.float32)
    blk = a512 // K
    mask1 = (blk[:, None] == blk[None, :]).astype(jnp.float32)
    w1big = mask1 * (proj @ W1 @ proj.T)
    mask2 = (blk[:, None] == jnp.arange(M)[None, :]).astype(jnp.float32)
    w2big = mask2 * (proj @ W2[:, :1])
    b1big = (proj @ b1).reshape(1, M * K)
    bmat, r = _prep(atom_fea, nbr_fea.reshape(N, M * K), Wtp,
                    w1big, b1big, w2big, b2[0].reshape(1, 1))
    out = _sc_gather(bmat, nbr_idx.astype(jnp.int32).reshape(NM),
                     r.reshape(NM))
    return out
